# baseline (speedup 1.0000x reference)
import functools

import jax
import jax.numpy as jnp
from jax import lax
from jax.experimental import pallas as pl
from jax.experimental.pallas import tpu as pltpu
from jax.experimental.pallas import tpu_sc as plsc

VOCAB_SIZE = 49408
HIDDEN_SIZE = 512
MAX_POS = 77
BATCH = 4096
SEQ = 77

NC = 2
NS = 16
NW = NC * NS
PER_W = BATCH // NW
NBUF = 2
LANES = 16
SEQ_PAD = 80
IDS_PAD = 128
NGROUP = SEQ_PAD // LANES

assert PER_W * NW == BATCH and PER_W % NBUF == 0

_mesh = plsc.VectorSubcoreMesh(core_axis_name="c", subcore_axis_name="s")


@functools.partial(
    pl.kernel,
    out_type=jax.ShapeDtypeStruct((BATCH, SEQ_PAD, HIDDEN_SIZE), jnp.float32),
    mesh=_mesh,
    scratch_types=[
        pltpu.VMEM((IDS_PAD,), jnp.int32),
        pltpu.VMEM((IDS_PAD,), jnp.int32),
        pltpu.VMEM((IDS_PAD,), jnp.int32),
        pltpu.VMEM((IDS_PAD,), jnp.int32),
        pltpu.VMEM((MAX_POS, HIDDEN_SIZE), jnp.float32),
        pltpu.VMEM((SEQ_PAD, HIDDEN_SIZE), jnp.float32),
        pltpu.VMEM((SEQ_PAD, HIDDEN_SIZE), jnp.float32),
        pltpu.SemaphoreType.DMA,
        pltpu.SemaphoreType.DMA,
        pltpu.SemaphoreType.DMA,
        pltpu.SemaphoreType.DMA,
        pltpu.SemaphoreType.DMA,
        pltpu.SemaphoreType.DMA,
    ],
)
def _emb_lookup(ids_hbm, pids_hbm, tok_hbm, pos_hbm, out_hbm,
                idxt0, idxt1, idxp0, idxp1, pos_v, buf0, buf1,
                semt0, semt1, semo0, semo1, semi0, semi1):
    wid = lax.axis_index("s") * NC + lax.axis_index("c")
    w_base = wid * PER_W

    bufs = (buf0, buf1)
    idxt = (idxt0, idxt1)
    idxp = (idxp0, idxp1)
    semt = (semt0, semt1)
    semo = (semo0, semo1)
    semi = (semi0, semi1)

    pltpu.sync_copy(pos_hbm, pos_v)

    def stage_t(g, b):
        return pltpu.make_async_copy(ids_hbm.at[w_base + g], idxt[b], semi[b])

    def stage_p(g, b):
        return pltpu.make_async_copy(pids_hbm.at[w_base + g], idxp[b], semi[b])

    def stage_start(g, b):
        stage_t(g, b).start()
        stage_p(g, b).start()

    def stage_wait(g, b):
        stage_t(g, b).wait()
        stage_p(g, b).wait()

    def tok_copy(g, b):
        return pltpu.make_async_copy(
            tok_hbm.at[idxt[b].at[pl.ds(0, SEQ_PAD)]], bufs[b], semt[b])

    def out_copy_a(g, b):
        return pltpu.make_async_copy(
            bufs[b].at[pl.ds(0, 48)],
            out_hbm.at[w_base + g].at[pl.ds(0, 48)], semo[b])

    def out_copy_b(g, b):
        return pltpu.make_async_copy(
            bufs[b].at[pl.ds(48, 32)],
            out_hbm.at[w_base + g].at[pl.ds(48, 32)], semo[b])

    def add_rows(b, klo, khi):
        def add_group(k, carry2):
            base = k * LANES
            p_vec = idxp[b][pl.ds(base, LANES)]
            for r16 in range(LANES):
                p_r = p_vec[r16]
                r = base + r16
                for j in range(HIDDEN_SIZE // LANES):
                    sl = pl.ds(j * LANES, LANES)
                    plsc.addupdate(bufs[b].at[r, sl], pos_v[p_r, sl])
            return carry2

        lax.fori_loop(klo, khi, add_group, 0)

    stage_start(0, 0)
    stage_wait(0, 0)
    tok_copy(0, 0).start()
    stage_start(1, 1)

    def superstep(kk, carry):
        for b in range(NBUF):
            g = NBUF * kk + b
            ob = 1 - b
            tok_copy(g, b).wait()

            @pl.when(g >= 1)
            def _():
                out_copy_a(g - 1, ob).wait()
                out_copy_b(g - 1, ob).wait()

            @pl.when(g + 1 < PER_W)
            def _():
                stage_wait(g + 1, ob)
                tok_copy(g + 1, ob).start()

            add_rows(b, 0, 3)
            out_copy_a(g, b).start()
            add_rows(b, 3, NGROUP)

            @pl.when(g + 2 < PER_W)
            def _():
                stage_start(g + 2, b)

            out_copy_b(g, b).start()
        return carry

    lax.fori_loop(0, PER_W // NBUF, superstep, 0)
    out_copy_a(PER_W - 1, (PER_W - 1) % NBUF).wait()
    out_copy_b(PER_W - 1, (PER_W - 1) % NBUF).wait()


def kernel(input_ids, position_ids, token_embedding, position_embedding):
    ids = jnp.pad(input_ids.astype(jnp.int32), ((0, 0), (0, IDS_PAD - SEQ)))
    pids = jnp.pad(position_ids.astype(jnp.int32), ((0, 0), (0, IDS_PAD - SEQ)))
    out = _emb_lookup(ids, pids, token_embedding, position_embedding)
    return out[:, :SEQ, :]
